# baseline (speedup 1.0000x reference)
import numpy as np
import jax
import jax.numpy as jnp
from jax.experimental import pallas as pl


def _ncg_logits_kernel(s_ref, g_ref, p_ref, o_ref):
    v = g_ref[...] * 0.5 + s_ref[...] - 0.5
    o_ref[...] = jax.lax.dot(v, p_ref[...], preferred_element_type=jnp.float32)


def kernel(s, g):
    B, D = s.shape
    bd = 512
    p = np.zeros((bd, 2 * bd), dtype=np.float32)
    p[np.arange(bd), 2 * np.arange(bd) + 1] = 1.0
    p = jnp.asarray(p)
    out = pl.pallas_call(
        _ncg_logits_kernel,
        grid=(D // bd,),
        in_specs=[
            pl.BlockSpec((B, bd), lambda j: (0, j)),
            pl.BlockSpec((B, bd), lambda j: (0, j)),
            pl.BlockSpec((bd, 2 * bd), lambda j: (0, 0)),
        ],
        out_specs=pl.BlockSpec((B, 2 * bd), lambda j: (0, j)),
        out_shape=jax.ShapeDtypeStruct((B, 2 * D), jnp.float32),
    )(s, g, p)
    return out.reshape(B, D, 2)


# speedup vs baseline: 1.1681x; 1.1681x over previous
import numpy as np
import jax
import jax.numpy as jnp
from jax.experimental import pallas as pl


def _ncg_logits_kernel(s_ref, g_ref, p_ref, o_ref):
    v = g_ref[...] * 0.5 + s_ref[...] - 0.5
    bb, bd = v.shape
    pm = p_ref[...]
    for k in range(bd // 128):
        o_ref[:, pl.ds(256 * k, 256)] = jax.lax.dot(
            v[:, 128 * k : 128 * (k + 1)], pm,
            preferred_element_type=jnp.float32,
        )


def kernel(s, g):
    B, D = s.shape
    bd = 1024
    p = np.zeros((128, 256), dtype=np.float32)
    p[np.arange(128), 2 * np.arange(128) + 1] = 1.0
    p = jnp.asarray(p)
    out = pl.pallas_call(
        _ncg_logits_kernel,
        grid=(D // bd,),
        in_specs=[
            pl.BlockSpec((B, bd), lambda j: (0, j)),
            pl.BlockSpec((B, bd), lambda j: (0, j)),
            pl.BlockSpec((128, 256), lambda j: (0, 0)),
        ],
        out_specs=pl.BlockSpec((B, 2 * bd), lambda j: (0, j)),
        out_shape=jax.ShapeDtypeStruct((B, 2 * D), jnp.float32),
    )(s, g, p)
    return out.reshape(B, D, 2)


# speedup vs baseline: 4.8915x; 4.1875x over previous
import jax
import jax.numpy as jnp
from jax.experimental import pallas as pl


def _ncg_logits_kernel(s_ref, g_ref, o_ref):
    v = g_ref[...] * 0.5 + s_ref[...] - 0.5
    o_ref[:, 0, :] = jnp.zeros_like(v)
    o_ref[:, 1, :] = v


def kernel(s, g):
    B, D = s.shape
    bd = 2048
    out = pl.pallas_call(
        _ncg_logits_kernel,
        grid=(D // bd,),
        in_specs=[
            pl.BlockSpec((B, bd), lambda j: (0, j)),
            pl.BlockSpec((B, bd), lambda j: (0, j)),
        ],
        out_specs=pl.BlockSpec((B, 2, bd), lambda j: (0, 0, j)),
        out_shape=jax.ShapeDtypeStruct((B, 2, D), jnp.float32),
    )(s, g)
    return out.swapaxes(1, 2)
